# Initial kernel scaffold; baseline (speedup 1.0000x reference)
import jax
import jax.numpy as jnp
from jax.experimental import pallas as pl


def kernel(x):
    raise NotImplementedError("write your pallas kernel here")


# speedup vs baseline: 28.5221x; 28.5221x over previous
import functools

import jax
import jax.numpy as jnp
from jax import lax
from jax.experimental import pallas as pl
from jax.experimental.pallas import tpu as pltpu
from jax.experimental.pallas import tpu_sc as plsc

B = 128
N = 32768
TOPK = 1638
L = 16
NC, NS = 2, 16
NW = NC * NS
ROWS_PER_W = B // NW
CHUNKS = N // L
NB1 = 512
INT_MIN = -(2 ** 31)


def _key(v):
    b = lax.bitcast_convert_type(v, jnp.int32)
    return b ^ (lax.shift_right_arithmetic(b, 31) | INT_MIN)


def _select_bin(hist_v, suf_v, nbins, nbits, kr):
    zeros = jnp.zeros((L,), jnp.int32)
    suf_v[pl.ds(nbins * L, L)] = zeros

    @plsc.parallel_loop(0, nbins, 1, carry=zeros)
    def _suf(i, run):
        bb = (nbins - 1 - i) * L
        run = run + hist_v[pl.ds(bb, L)]
        suf_v[pl.ds(bb, L)] = run
        hist_v[pl.ds(bb, L)] = zeros
        return run

    def search_body(_, lohi):
        lo, hi = lohi
        mid = (lo + hi) // 2
        good = jnp.sum(suf_v[pl.ds(mid * L, L)]) >= kr
        return (jnp.where(good, mid, lo), jnp.where(good, hi, mid))

    lo, _ = lax.fori_loop(0, nbits, search_body,
                          (jnp.int32(0), jnp.int32(nbins)))
    cnt_gt = jnp.sum(suf_v[pl.ds((lo + 1) * L, L)])
    return lo, kr - cnt_gt


def kernel(x):
    mesh = plsc.VectorSubcoreMesh(core_axis_name="c", subcore_axis_name="s")

    @functools.partial(
        pl.kernel,
        out_type=jax.ShapeDtypeStruct((B, N), jnp.float32),
        mesh=mesh,
        scratch_types=[
            pltpu.VMEM((N,), jnp.float32),
            pltpu.VMEM((N,), jnp.float32),
            pltpu.VMEM((N,), jnp.int32),
            pltpu.VMEM((NB1 * L,), jnp.int32),
            pltpu.VMEM(((NB1 + 1) * L,), jnp.int32),
            pltpu.SemaphoreType.DMA,
            pltpu.SemaphoreType.DMA,
        ],
        compiler_params=pltpu.CompilerParams(needs_layout_passes=False),
    )
    def kwta(x_hbm, out_hbm, rowa_v, rowb_v, cand_v, hist_v, suf_v,
             in_sem, out_sem):
        wid = lax.axis_index("s") * NC + lax.axis_index("c")
        iota = lax.iota(jnp.int32, L)
        ones = jnp.ones((L,), jnp.int32)
        zeros = jnp.zeros((L,), jnp.int32)

        @plsc.parallel_loop(0, NB1, 1, unroll=8)
        def _zero(i):
            hist_v[pl.ds(i * L, L)] = zeros

        def process(row_v):
            @plsc.parallel_loop(0, CHUNKS, 1, unroll=8)
            def _p1(i):
                u = _key(row_v[pl.ds(i * L, L)])
                idx = (lax.shift_right_logical(u, 19) & (0x1FF << 4)) | iota
                plsc.addupdate_scatter(hist_v, [idx], ones)

            b1, kr = _select_bin(hist_v, suf_v, 512, 9, jnp.int32(TOPK))

            @plsc.parallel_loop(0, CHUNKS, 1, unroll=4, carry=zeros)
            def _c1(i, off_vec):
                u = _key(row_v[pl.ds(i * L, L)])
                pm = lax.shift_right_logical(u, 23) == b1
                pm_i = jnp.where(pm, 1, 0)
                dst = off_vec + plsc.cumsum(pm_i) - pm_i
                plsc.store_scatter(cand_v, [dst], u, mask=pm)
                return off_vec + plsc.all_reduce_population_count(pm)

            n_c = jnp.max(_c1)
            nc_chunks = lax.div(n_c + (L - 1), jnp.int32(L))

            def p2_body(i, _):
                u = cand_v[pl.ds(i * L, L)]
                lm = (i * L + iota) < n_c
                idx = (lax.shift_right_logical(u, 11) & (0xFF << 4)) | iota
                plsc.addupdate_scatter(hist_v, [idx], ones, mask=lm)
                return 0

            lax.fori_loop(0, nc_chunks, p2_body, 0)
            b2, kr = _select_bin(hist_v, suf_v, 256, 8, kr)

            def c2_body(i, off_vec):
                u = cand_v[pl.ds(i * L, L)]
                lm = (i * L + iota) < n_c
                pm = ((lax.shift_right_logical(u, 15) & 0xFF) == b2) & lm
                pm_i = jnp.where(pm, 1, 0)
                dst = off_vec + plsc.cumsum(pm_i) - pm_i
                plsc.store_scatter(cand_v, [dst], u, mask=pm)
                return off_vec + plsc.all_reduce_population_count(pm)

            off2 = lax.fori_loop(0, nc_chunks, c2_body, zeros)
            n_c2 = jnp.max(off2)
            nc2_chunks = lax.div(n_c2 + (L - 1), jnp.int32(L))

            def bit_step(t_low, bit):
                t = t_low | (1 << bit)

                def cnt_body(i, cnt_vec):
                    u = cand_v[pl.ds(i * L, L)]
                    lm = (i * L + iota) < n_c2
                    ge = ((u & 0x7FFF) >= t) & lm
                    return cnt_vec + jnp.where(ge, 1, 0)

                cnt = jnp.sum(lax.fori_loop(0, nc2_chunks, cnt_body, zeros))
                return jnp.where(cnt >= kr, t, t_low)

            t_low = jnp.int32(0)
            for bit in range(14, -1, -1):
                t_low = bit_step(t_low, bit)

            tfull = (((b1 << 8) | b2) << 15) | t_low
            ts = tfull ^ INT_MIN

            @plsc.parallel_loop(0, CHUNKS, 1, unroll=8)
            def _fin(i):
                sl = pl.ds(i * L, L)
                v = row_v[sl]
                b = lax.bitcast_convert_type(v, jnp.int32)
                s = b ^ lax.shift_right_logical(
                    lax.shift_right_arithmetic(b, 31), 1)
                row_v[sl] = jnp.where(s >= ts, v, jnp.float32(0.0))

        bufs = (rowa_v, rowb_v)
        rows = [wid * ROWS_PER_W + r for r in range(ROWS_PER_W)]
        in_d, out_d = {}, {}
        in_d[0] = pltpu.async_copy(x_hbm.at[rows[0]], bufs[0], in_sem)
        for r in range(ROWS_PER_W):
            buf = bufs[r % 2]
            in_d[r].wait()
            if r + 1 < ROWS_PER_W:
                if r >= 1:
                    out_d[r - 1].wait()
                in_d[r + 1] = pltpu.async_copy(
                    x_hbm.at[rows[r + 1]], bufs[(r + 1) % 2], in_sem)
            process(buf)
            out_d[r] = pltpu.async_copy(buf, out_hbm.at[rows[r]], out_sem)
        out_d[ROWS_PER_W - 2].wait()
        out_d[ROWS_PER_W - 1].wait()

    return kwta(x)
